# baseline (speedup 1.0000x reference)
import functools

import jax
import jax.numpy as jnp
from jax import lax
from jax.experimental import pallas as pl
from jax.experimental.pallas import tpu as pltpu
from jax.experimental.pallas import tpu_sc as plsc

N = 10000
E = 320000
D = 128

NC = 2
NS = 16
NT = NC * NS
BLK = 128
NBUF = 4
NB = 80
CAP = NT * NB * BLK
NROWS = CAP // BLK
EPT = NB * BLK
EPT2 = E // NT
NBC = NB + 1
CAPT = NBC * BLK
ACC_ROWS = 10240
NQ = 4
QROWS = ACC_ROWS // NQ
NPH = NQ // NC
ROWS_PER_TILE = QROWS // NS
ZROWS = 32
IGN = -1

ROW_BLK = 2048
PREP_GRID = 8
PREP_COLS = CAP // PREP_GRID

_mesh = plsc.VectorSubcoreMesh(
    core_axis_name="c", subcore_axis_name="s", num_cores=NC, num_subcores=NS)


_COMPACT_KW = dict(
    out_type=[
        jax.ShapeDtypeStruct((NT, ACC_ROWS), jnp.float32),
        jax.ShapeDtypeStruct((NT, NQ, CAPT), jnp.int32),
        jax.ShapeDtypeStruct((NT, NQ, CAPT), jnp.int32),
        jax.ShapeDtypeStruct((NT, 16), jnp.float32),
    ],
    mesh=_mesh,
    scratch_types=[
        pltpu.VMEM((EPT2,), jnp.int32),
        pltpu.VMEM((EPT2,), jnp.int32),
        pltpu.VMEM((NQ * CAPT,), jnp.int32),
        pltpu.VMEM((NQ * CAPT,), jnp.int32),
        pltpu.VMEM((ACC_ROWS,), jnp.float32),
        pltpu.VMEM((16,), jnp.float32),
    ],
    compiler_params=pltpu.CompilerParams(needs_layout_passes=False),
)


def _compact_body(ei_hbm, deg_hbm, csrc_hbm, cdst_hbm, cnt_hbm,
                  src_raw, dst_raw, csrc, cdst, deg, cnt_v):
    c = lax.axis_index("c")
    s = lax.axis_index("s")
    wid = c * NS + s
    pltpu.sync_copy(ei_hbm.at[pl.ds(wid * EPT2, EPT2)], src_raw)
    pltpu.sync_copy(ei_hbm.at[pl.ds(E + wid * EPT2, EPT2)], dst_raw)

    def zero_body(i, carry):
        deg[pl.ds(i * 16, 16)] = jnp.zeros((16,), jnp.float32)
        return carry

    lax.fori_loop(0, ACC_ROWS // 16, zero_body, 0)

    ones = jnp.ones((16,), jnp.float32)
    lane = lax.iota(jnp.int32, 16)

    def scan_body(i, offs):
        sv = src_raw[pl.ds(i * 16, 16)]
        dv = dst_raw[pl.ds(i * 16, 16)]
        plsc.addupdate_scatter(deg, [dv], ones)
        new_offs = []
        for q in range(NQ):
            m = (dv >= q * QROWS) & (dv < (q + 1) * QROWS)
            pos = q * CAPT + offs[q] + plsc.cumsum(m.astype(jnp.int32)) - 1
            plsc.store_scatter(csrc, [pos], sv, mask=m)
            plsc.store_scatter(cdst, [pos], dv - q * QROWS, mask=m)
            new_offs.append(offs[q] + plsc.all_reduce_population_count(m))
        return tuple(new_offs)

    zero_v = jnp.zeros((16,), jnp.int32)
    offs = lax.fori_loop(0, EPT2 // 16, scan_body,
                         (zero_v, zero_v, zero_v, zero_v))

    ign_v = jnp.full((16,), IGN, jnp.int32)
    for q in range(NQ):
        for k in range(BLK // 16):
            pos = q * CAPT + offs[q] + lane + k * 16
            plsc.store_scatter(csrc, [pos], ign_v)
            plsc.store_scatter(cdst, [pos], ign_v)

    cv = jnp.zeros((16,), jnp.float32)
    for q in range(NQ):
        cv = jnp.where(lane == q, offs[q].astype(jnp.float32), cv)
    cnt_v[...] = cv

    pltpu.sync_copy(deg, deg_hbm.at[wid])
    for q in range(NQ):
        pltpu.sync_copy(csrc.at[pl.ds(q * CAPT, CAPT)], csrc_hbm.at[wid, q])
        pltpu.sync_copy(cdst.at[pl.ds(q * CAPT, CAPT)], cdst_hbm.at[wid, q])
    pltpu.sync_copy(cnt_v, cnt_hbm.at[wid])


_compact_kernel = pl.kernel(_compact_body, **_COMPACT_KW)


def _hw_body(h_ref, w_ref, out_ref):
    out_ref[...] = jnp.dot(h_ref[...], w_ref[...],
                           preferred_element_type=jnp.float32)


def _hw_call(h, W):
    return pl.pallas_call(
        _hw_body,
        grid=(ACC_ROWS // ROW_BLK,),
        in_specs=[
            pl.BlockSpec((ROW_BLK, D), lambda i: (i, 0)),
            pl.BlockSpec((D, D), lambda i: (0, 0)),
        ],
        out_specs=pl.BlockSpec((ROW_BLK, D), lambda i: (i, 0)),
        out_shape=jax.ShapeDtypeStruct((N, D), jnp.float32),
    )(h, W)


def _scale_body(hw_ref, degp_ref, out_ref):
    deg = jnp.sum(degp_ref[...], axis=0)
    norm = lax.rsqrt(jnp.maximum(deg, 1.0))
    out_ref[...] = hw_ref[...] * norm[:, None]


def _mm_call(hw, degp):
    return pl.pallas_call(
        _scale_body,
        grid=(ACC_ROWS // ROW_BLK,),
        in_specs=[
            pl.BlockSpec((ROW_BLK, D), lambda i: (i, 0)),
            pl.BlockSpec((NT, ROW_BLK), lambda i: (0, i)),
        ],
        out_specs=pl.BlockSpec((ROW_BLK, D), lambda i: (i, 0)),
        out_shape=jax.ShapeDtypeStruct((N, D), jnp.float32),
    )(hw, degp)


_SCATTER_KW = dict(
    out_type=jax.ShapeDtypeStruct((NQ, QROWS, D), jnp.float32),
    mesh=_mesh,
    scratch_types=[
        pltpu.VMEM((NBC, BLK), jnp.int32),
        pltpu.VMEM((NBC, BLK), jnp.int32),
        [pltpu.VMEM((BLK, D), jnp.float32) for _ in range(NBUF)],
        pltpu.VMEM((ZROWS, D), jnp.float32),
        pltpu.VMEM((16,), jnp.float32),
        pltpu.VMEM_SHARED((QROWS, D), jnp.float32),
        [pltpu.SemaphoreType.DMA for _ in range(NBUF)],
    ],
    compiler_params=pltpu.CompilerParams(needs_layout_passes=False),
)


def _scatter_body(csrc_hbm, cdst_hbm, cnt_hbm, hwn_hbm, out_hbm,
                  src_idx, dst_idx, bufs, zbuf, cnt_v, acc, sems):
    c = lax.axis_index("c")
    s = lax.axis_index("s")
    base = s * ROWS_PER_TILE

    def zb(i, carry):
        zbuf[i // (D // 16), pl.ds((i % (D // 16)) * 16, 16)] = (
            jnp.zeros((16,), jnp.float32))
        return carry

    lax.fori_loop(0, ZROWS * (D // 16), zb, 0)

    def _gather(j, b):
        return pltpu.make_async_copy(
            hwn_hbm.at[plsc.Indices(src_idx.at[j], ignored_value=IGN)],
            bufs[b], sems[b])

    def _scatter_add(j, b):
        pltpu.sync_copy(
            bufs[b],
            acc.at[plsc.Indices(dst_idx.at[j], ignored_value=IGN)],
            add=True)

    for p in range(NPH):
        q = c * NPH + p
        if p > 0:
            plsc.subcore_barrier()
        for m in range(ROWS_PER_TILE // ZROWS):
            pltpu.sync_copy(zbuf, acc.at[pl.ds(base + m * ZROWS, ZROWS)])
        plsc.subcore_barrier()

        for sub in range(NT // NS):
            prod = s * (NT // NS) + sub
            pltpu.sync_copy(csrc_hbm.at[prod, q], src_idx)
            pltpu.sync_copy(cdst_hbm.at[prod, q], dst_idx)
            pltpu.sync_copy(cnt_hbm.at[prod], cnt_v)
            lane = lax.iota(jnp.int32, 16)
            cnt = jnp.sum(
                jnp.where(lane == q, cnt_v[...], 0.0)).astype(jnp.int32)
            nblk = jnp.clip((cnt + BLK - 1) // BLK, 0, NBC)

            for b in range(NBUF):
                @pl.when(b < nblk)
                def _():
                    _gather(b, b).start()

            def loop_body(g, carry):
                for b in range(NBUF):
                    j = g * NBUF + b

                    @pl.when(j < nblk)
                    def _():
                        _gather(j, b).wait()
                        _scatter_add(j, b)

                        @pl.when(j + NBUF < nblk)
                        def _():
                            _gather(j + NBUF, b).start()
                return carry

            lax.fori_loop(0, (nblk + NBUF - 1) // NBUF, loop_body, 0)

        plsc.subcore_barrier()
        pltpu.sync_copy(acc.at[pl.ds(base, ROWS_PER_TILE)],
                        out_hbm.at[q, pl.ds(base, ROWS_PER_TILE)])


_scatter_kernel = pl.kernel(_scatter_body, **_SCATTER_KW)


def _ep_body(acc_ref, degp_ref, b_ref, out_ref):
    deg = jnp.sum(degp_ref[...], axis=0)
    norm = lax.rsqrt(jnp.maximum(deg, 1.0))
    out_ref[...] = jnp.maximum(acc_ref[...] * norm[:, None] + b_ref[...], 0.0)


def _ep_call(acc, degp, b2d):
    return pl.pallas_call(
        _ep_body,
        grid=(ACC_ROWS // ROW_BLK,),
        in_specs=[
            pl.BlockSpec((ROW_BLK, D), lambda i: (i, 0)),
            pl.BlockSpec((NT, ROW_BLK), lambda i: (0, i)),
            pl.BlockSpec((1, D), lambda i: (0, 0)),
        ],
        out_specs=pl.BlockSpec((ROW_BLK, D), lambda i: (i, 0)),
        out_shape=jax.ShapeDtypeStruct((N, D), jnp.float32),
    )(acc, degp, b2d)


def kernel(h, edge_index, W, b):
    hw = _hw_call(h, W)
    degp, csrc, cdst, cnts = _compact_kernel(edge_index.reshape(2 * E))
    hwn = _mm_call(hw, degp)
    accp = _scatter_kernel(csrc.reshape(NT, NQ, NBC, BLK),
                           cdst.reshape(NT, NQ, NBC, BLK), cnts, hwn)
    return _ep_call(accp.reshape(ACC_ROWS, D), degp, b.reshape(1, D))


# speedup vs baseline: 1.0064x; 1.0064x over previous
import functools

import jax
import jax.numpy as jnp
from jax import lax
from jax.experimental import pallas as pl
from jax.experimental.pallas import tpu as pltpu
from jax.experimental.pallas import tpu_sc as plsc

N = 10000
E = 320000
D = 128

NC = 2
NS = 16
NT = NC * NS
BLK = 128
NBUF = 4
NB = 80
CAP = NT * NB * BLK
NROWS = CAP // BLK
EPT = NB * BLK
EPT2 = E // NT
NBC = NB + 1
CAPT = NBC * BLK
ACC_ROWS = 10240
NQ = 4
QROWS = ACC_ROWS // NQ
NPH = NQ // NC
ROWS_PER_TILE = QROWS // NS
ZROWS = 32
IGN = -1

ROW_BLK = 2048
PREP_GRID = 8
PREP_COLS = CAP // PREP_GRID

_mesh = plsc.VectorSubcoreMesh(
    core_axis_name="c", subcore_axis_name="s", num_cores=NC, num_subcores=NS)


_COMPACT_KW = dict(
    out_type=[
        jax.ShapeDtypeStruct((NT, ACC_ROWS), jnp.float32),
        jax.ShapeDtypeStruct((NT, NQ, CAPT), jnp.int32),
        jax.ShapeDtypeStruct((NT, NQ, CAPT), jnp.int32),
        jax.ShapeDtypeStruct((NT, 16), jnp.float32),
    ],
    mesh=_mesh,
    scratch_types=[
        pltpu.VMEM((EPT2,), jnp.int32),
        pltpu.VMEM((EPT2,), jnp.int32),
        pltpu.VMEM((NQ * CAPT,), jnp.int32),
        pltpu.VMEM((NQ * CAPT,), jnp.int32),
        pltpu.VMEM((ACC_ROWS,), jnp.float32),
        pltpu.VMEM((16,), jnp.float32),
    ],
    compiler_params=pltpu.CompilerParams(needs_layout_passes=False),
)


def _compact_body(ei_hbm, deg_hbm, csrc_hbm, cdst_hbm, cnt_hbm,
                  src_raw, dst_raw, csrc, cdst, deg, cnt_v):
    c = lax.axis_index("c")
    s = lax.axis_index("s")
    wid = c * NS + s
    pltpu.sync_copy(ei_hbm.at[pl.ds(wid * EPT2, EPT2)], src_raw)
    pltpu.sync_copy(ei_hbm.at[pl.ds(E + wid * EPT2, EPT2)], dst_raw)

    def zero_body(i, carry):
        deg[pl.ds(i * 16, 16)] = jnp.zeros((16,), jnp.float32)
        return carry

    lax.fori_loop(0, ACC_ROWS // 16, zero_body, 0)

    ones = jnp.ones((16,), jnp.float32)
    lane = lax.iota(jnp.int32, 16)

    def scan_body(i, offs):
        sv = src_raw[pl.ds(i * 16, 16)]
        dv = dst_raw[pl.ds(i * 16, 16)]
        plsc.addupdate_scatter(deg, [dv], ones)
        new_offs = []
        for q in range(NQ):
            m = (dv >= q * QROWS) & (dv < (q + 1) * QROWS)
            pos = q * CAPT + offs[q] + plsc.cumsum(m.astype(jnp.int32)) - 1
            plsc.store_scatter(csrc, [pos], sv, mask=m)
            plsc.store_scatter(cdst, [pos], dv - q * QROWS, mask=m)
            new_offs.append(offs[q] + plsc.all_reduce_population_count(m))
        return tuple(new_offs)

    zero_v = jnp.zeros((16,), jnp.int32)
    offs = lax.fori_loop(0, EPT2 // 16, scan_body,
                         (zero_v, zero_v, zero_v, zero_v))

    ign_v = jnp.full((16,), IGN, jnp.int32)
    for q in range(NQ):
        for k in range(BLK // 16):
            pos = q * CAPT + offs[q] + lane + k * 16
            plsc.store_scatter(csrc, [pos], ign_v)
            plsc.store_scatter(cdst, [pos], ign_v)

    cv = jnp.zeros((16,), jnp.float32)
    for q in range(NQ):
        cv = jnp.where(lane == q, offs[q].astype(jnp.float32), cv)
    cnt_v[...] = cv

    pltpu.sync_copy(deg, deg_hbm.at[wid])
    for q in range(NQ):
        pltpu.sync_copy(csrc.at[pl.ds(q * CAPT, CAPT)], csrc_hbm.at[wid, q])
        pltpu.sync_copy(cdst.at[pl.ds(q * CAPT, CAPT)], cdst_hbm.at[wid, q])
    pltpu.sync_copy(cnt_v, cnt_hbm.at[wid])


_compact_kernel = pl.kernel(_compact_body, **_COMPACT_KW)


def _mm_body(h_ref, w_ref, degp_ref, out_ref):
    hw = jnp.dot(h_ref[...], w_ref[...], preferred_element_type=jnp.float32)
    deg = jnp.sum(degp_ref[...], axis=0)
    norm = lax.rsqrt(jnp.maximum(deg, 1.0))
    out_ref[...] = hw * norm[:, None]


def _mm_call(h, W, degp):
    return pl.pallas_call(
        _mm_body,
        grid=(ACC_ROWS // ROW_BLK,),
        in_specs=[
            pl.BlockSpec((ROW_BLK, D), lambda i: (i, 0)),
            pl.BlockSpec((D, D), lambda i: (0, 0)),
            pl.BlockSpec((NT, ROW_BLK), lambda i: (0, i)),
        ],
        out_specs=pl.BlockSpec((ROW_BLK, D), lambda i: (i, 0)),
        out_shape=jax.ShapeDtypeStruct((N, D), jnp.float32),
    )(h, W, degp)


_SCATTER_KW = dict(
    out_type=jax.ShapeDtypeStruct((NQ, QROWS, D), jnp.float32),
    mesh=_mesh,
    scratch_types=[
        pltpu.VMEM((NBC, BLK), jnp.int32),
        pltpu.VMEM((NBC, BLK), jnp.int32),
        [pltpu.VMEM((BLK, D), jnp.float32) for _ in range(NBUF)],
        pltpu.VMEM((ZROWS, D), jnp.float32),
        pltpu.VMEM((16,), jnp.float32),
        pltpu.VMEM_SHARED((QROWS, D), jnp.float32),
        [pltpu.SemaphoreType.DMA for _ in range(NBUF)],
    ],
    compiler_params=pltpu.CompilerParams(needs_layout_passes=False),
)


def _scatter_body(csrc_hbm, cdst_hbm, cnt_hbm, hwn_hbm, out_hbm,
                  src_idx, dst_idx, bufs, zbuf, cnt_v, acc, sems):
    c = lax.axis_index("c")
    s = lax.axis_index("s")
    base = s * ROWS_PER_TILE

    def zb(i, carry):
        zbuf[i // (D // 16), pl.ds((i % (D // 16)) * 16, 16)] = (
            jnp.zeros((16,), jnp.float32))
        return carry

    lax.fori_loop(0, ZROWS * (D // 16), zb, 0)

    def _gather(j, b):
        return pltpu.make_async_copy(
            hwn_hbm.at[plsc.Indices(src_idx.at[j], ignored_value=IGN)],
            bufs[b], sems[b])

    def _scatter_add(j, b):
        pltpu.sync_copy(
            bufs[b],
            acc.at[plsc.Indices(dst_idx.at[j], ignored_value=IGN)],
            add=True)

    for p in range(NPH):
        q = c * NPH + p
        if p > 0:
            plsc.subcore_barrier()
        for m in range(ROWS_PER_TILE // ZROWS):
            pltpu.sync_copy(zbuf, acc.at[pl.ds(base + m * ZROWS, ZROWS)])
        plsc.subcore_barrier()

        for sub in range(NT // NS):
            prod = s * (NT // NS) + sub
            pltpu.sync_copy(csrc_hbm.at[prod, q], src_idx)
            pltpu.sync_copy(cdst_hbm.at[prod, q], dst_idx)
            pltpu.sync_copy(cnt_hbm.at[prod], cnt_v)
            lane = lax.iota(jnp.int32, 16)
            cnt = jnp.sum(
                jnp.where(lane == q, cnt_v[...], 0.0)).astype(jnp.int32)
            nblk = jnp.clip((cnt + BLK - 1) // BLK, 0, NBC)

            for b in range(NBUF):
                @pl.when(b < nblk)
                def _():
                    _gather(b, b).start()

            def loop_body(g, carry):
                for b in range(NBUF):
                    j = g * NBUF + b

                    @pl.when(j < nblk)
                    def _():
                        _gather(j, b).wait()
                        _scatter_add(j, b)

                        @pl.when(j + NBUF < nblk)
                        def _():
                            _gather(j + NBUF, b).start()
                return carry

            lax.fori_loop(0, (nblk + NBUF - 1) // NBUF, loop_body, 0)

        plsc.subcore_barrier()
        pltpu.sync_copy(acc.at[pl.ds(base, ROWS_PER_TILE)],
                        out_hbm.at[q, pl.ds(base, ROWS_PER_TILE)])


_scatter_kernel = pl.kernel(_scatter_body, **_SCATTER_KW)


def _ep_body(acc_ref, degp_ref, b_ref, out_ref):
    deg = jnp.sum(degp_ref[...], axis=0)
    norm = lax.rsqrt(jnp.maximum(deg, 1.0))
    out_ref[...] = jnp.maximum(acc_ref[...] * norm[:, None] + b_ref[...], 0.0)


def _ep_call(acc, degp, b2d):
    return pl.pallas_call(
        _ep_body,
        grid=(ACC_ROWS // ROW_BLK,),
        in_specs=[
            pl.BlockSpec((ROW_BLK, D), lambda i: (i, 0)),
            pl.BlockSpec((NT, ROW_BLK), lambda i: (0, i)),
            pl.BlockSpec((1, D), lambda i: (0, 0)),
        ],
        out_specs=pl.BlockSpec((ROW_BLK, D), lambda i: (i, 0)),
        out_shape=jax.ShapeDtypeStruct((N, D), jnp.float32),
    )(acc, degp, b2d)


def kernel(h, edge_index, W, b):
    degp, csrc, cdst, cnts = _compact_kernel(edge_index.reshape(2 * E))
    hwn = _mm_call(h, W, degp)
    accp = _scatter_kernel(csrc.reshape(NT, NQ, NBC, BLK),
                           cdst.reshape(NT, NQ, NBC, BLK), cnts, hwn)
    return _ep_call(accp.reshape(ACC_ROWS, D), degp, b.reshape(1, D))
